# baseline (speedup 1.0000x reference)
import jax
import jax.numpy as jnp
from jax.experimental import pallas as pl
from jax.experimental.pallas import tpu as pltpu

B = 4
N = 8192
H = 768
C = 256
K = 32

TN = 1024
NT = N // TN


def _body(x_ref, we_ref, be_ref, wa_ref, ba_ref, wd_ref, bd_ref,
          u_ref, z_ref, f_ref, uscr, psum, zscr):
    b = pl.program_id(0)
    p = pl.program_id(1)
    n = pl.program_id(2)

    @pl.when(p == 0)
    def _encode():
        xt = x_ref[0]
        u = jnp.maximum(
            jnp.dot(xt, we_ref[...], preferred_element_type=jnp.float32)
            + be_ref[...], 0.0)
        u_ref[0] = u
        uscr[pl.ds(n * TN, TN), :] = u
        part = jnp.sum(u, axis=0, keepdims=True)

        @pl.when(n == 0)
        def _():
            psum[...] = part

        @pl.when(n != 0)
        def _():
            psum[...] += part

    @pl.when(p == 1)
    def _decode():
        @pl.when(n == 0)
        def _topk():
            u_bar = psum[...] * (1.0 / N)
            logits = jnp.dot(u_bar, wa_ref[...],
                             preferred_element_type=jnp.float32) + ba_ref[...]
            zrow = jax.nn.sigmoid(logits)
            zcol = jnp.transpose(zrow)
            jiota = jax.lax.broadcasted_iota(jnp.int32, (C, C), 0)
            iiota = jax.lax.broadcasted_iota(jnp.int32, (C, C), 1)
            g = (zcol > zrow) | ((zcol == zrow) & (jiota < iiota))
            cnt = jnp.sum(g.astype(jnp.int32), axis=0, keepdims=True)
            zmask = zrow * (cnt < K).astype(jnp.float32)
            zscr[...] = zmask
            z_ref[pl.ds(b, 1), :] = zmask

        ut = uscr[pl.ds(n * TN, TN), :]
        gated = ut * zscr[...]
        f_ref[0] = jnp.dot(gated, wd_ref[...],
                           preferred_element_type=jnp.float32) + bd_ref[...]


@jax.jit
def kernel(x, W_enc, b_enc, W_agg, b_agg, W_dec, b_dec):
    b_enc2 = b_enc.reshape(1, C)
    b_agg2 = b_agg.reshape(1, C)
    b_dec2 = b_dec.reshape(1, H)

    u, z_hat, f_hat = pl.pallas_call(
        _body,
        grid=(B, 2, NT),
        in_specs=[
            pl.BlockSpec((1, TN, H),
                         lambda b, p, n: (b, jnp.where(p == 0, n, NT - 1), 0)),
            pl.BlockSpec((H, C), lambda b, p, n: (0, 0)),
            pl.BlockSpec((1, C), lambda b, p, n: (0, 0)),
            pl.BlockSpec((C, C), lambda b, p, n: (0, 0)),
            pl.BlockSpec((1, C), lambda b, p, n: (0, 0)),
            pl.BlockSpec((C, H), lambda b, p, n: (0, 0)),
            pl.BlockSpec((1, H), lambda b, p, n: (0, 0)),
        ],
        out_specs=[
            pl.BlockSpec((1, TN, C),
                         lambda b, p, n: (b, jnp.where(p == 0, n, NT - 1), 0)),
            pl.BlockSpec((B, C), lambda b, p, n: (0, 0)),
            pl.BlockSpec((1, TN, H),
                         lambda b, p, n: (b, jnp.where(p == 0, 0, n), 0)),
        ],
        out_shape=[
            jax.ShapeDtypeStruct((B, N, C), jnp.float32),
            jax.ShapeDtypeStruct((B, C), jnp.float32),
            jax.ShapeDtypeStruct((B, N, H), jnp.float32),
        ],
        scratch_shapes=[
            pltpu.VMEM((N, C), jnp.float32),
            pltpu.VMEM((1, C), jnp.float32),
            pltpu.VMEM((1, C), jnp.float32),
        ],
        compiler_params=pltpu.CompilerParams(
            dimension_semantics=("arbitrary", "arbitrary", "arbitrary"),
        ),
    )(x, W_enc, b_enc2, W_agg, b_agg2, W_dec, b_dec2)

    return (f_hat, z_hat, u)


# speedup vs baseline: 1.2401x; 1.2401x over previous
import jax
import jax.numpy as jnp
from jax.experimental import pallas as pl
from jax.experimental.pallas import tpu as pltpu

B = 4
N = 8192
H = 768
C = 256
K = 32

TN = 2048
NT = N // TN


def _body(x_ref, we_ref, be_ref, wa_ref, ba_ref, wd_ref, bd_ref,
          u_ref, z_ref, f_ref, uscr, psum, zscr):
    s = pl.program_id(0)
    n = pl.program_id(1)
    par = jax.lax.rem(s, 2)

    @pl.when(s > 0)
    def _decode():
        @pl.when(n == 0)
        def _topk():
            u_bar = psum[1 - par] * (1.0 / N)
            logits = jnp.dot(u_bar, wa_ref[...],
                             preferred_element_type=jnp.float32) + ba_ref[...]
            zrow = jax.nn.sigmoid(logits)
            zcol = jnp.transpose(zrow)
            jiota = jax.lax.broadcasted_iota(jnp.int32, (C, C), 0)
            iiota = jax.lax.broadcasted_iota(jnp.int32, (C, C), 1)
            g = (zcol > zrow) | ((zcol == zrow) & (jiota < iiota))
            cnt = jnp.sum(g.astype(jnp.int32), axis=0, keepdims=True)
            zmask = zrow * (cnt < K).astype(jnp.float32)
            zscr[...] = zmask
            z_ref[pl.ds(s - 1, 1), :] = zmask

        ut = uscr[1 - par, pl.ds(n * TN, TN), :]
        gated = ut * zscr[...]
        f_ref[0] = jnp.dot(gated, wd_ref[...],
                           preferred_element_type=jnp.float32) + bd_ref[...]

    @pl.when(s < B)
    def _encode():
        xt = x_ref[0]
        u = jnp.maximum(
            jnp.dot(xt, we_ref[...], preferred_element_type=jnp.float32)
            + be_ref[...], 0.0)
        u_ref[0] = u
        uscr[par, pl.ds(n * TN, TN), :] = u
        part = jnp.sum(u, axis=0, keepdims=True)

        @pl.when(n == 0)
        def _():
            psum[par] = part

        @pl.when(n != 0)
        def _():
            psum[par] += part


@jax.jit
def kernel(x, W_enc, b_enc, W_agg, b_agg, W_dec, b_dec):
    b_enc2 = b_enc.reshape(1, C)
    b_agg2 = b_agg.reshape(1, C)
    b_dec2 = b_dec.reshape(1, H)

    u, z_hat, f_hat = pl.pallas_call(
        _body,
        grid=(B + 1, NT),
        in_specs=[
            pl.BlockSpec((1, TN, H),
                         lambda s, n: (jnp.minimum(s, B - 1),
                                       jnp.where(s < B, n, NT - 1), 0)),
            pl.BlockSpec((H, C), lambda s, n: (0, 0)),
            pl.BlockSpec((1, C), lambda s, n: (0, 0)),
            pl.BlockSpec((C, C), lambda s, n: (0, 0)),
            pl.BlockSpec((1, C), lambda s, n: (0, 0)),
            pl.BlockSpec((C, H), lambda s, n: (0, 0)),
            pl.BlockSpec((1, H), lambda s, n: (0, 0)),
        ],
        out_specs=[
            pl.BlockSpec((1, TN, C),
                         lambda s, n: (jnp.minimum(s, B - 1),
                                       jnp.where(s < B, n, NT - 1), 0)),
            pl.BlockSpec((B, C), lambda s, n: (0, 0)),
            pl.BlockSpec((1, TN, H),
                         lambda s, n: (jnp.maximum(s - 1, 0),
                                       jnp.where(s > 0, n, 0), 0)),
        ],
        out_shape=[
            jax.ShapeDtypeStruct((B, N, C), jnp.float32),
            jax.ShapeDtypeStruct((B, C), jnp.float32),
            jax.ShapeDtypeStruct((B, N, H), jnp.float32),
        ],
        scratch_shapes=[
            pltpu.VMEM((2, N, C), jnp.float32),
            pltpu.VMEM((2, 1, C), jnp.float32),
            pltpu.VMEM((1, C), jnp.float32),
        ],
        compiler_params=pltpu.CompilerParams(
            dimension_semantics=("arbitrary", "arbitrary"),
        ),
    )(x, W_enc, b_enc2, W_agg, b_agg2, W_dec, b_dec2)

    return (f_hat, z_hat, u)
